# Initial kernel scaffold; baseline (speedup 1.0000x reference)
import functools

import jax
import jax.numpy as jnp
from jax import lax
from jax.experimental import pallas as pl
from jax.experimental.pallas import tpu as pltpu
from jax.experimental.pallas import tpu_sc as plsc

D = 64
C = 128
NC = 2
NS = 16
NW = NC * NS


def _emb_body(idx_hbm, tab_hbm, out_hbm, idx_v, rows_v, gsem):
    wid = lax.axis_index("s") * NC + lax.axis_index("c")
    cpw = idx_hbm.shape[0] // NW
    blk0 = wid * cpw
    pltpu.sync_copy(idx_hbm.at[pl.ds(blk0, cpw)], idx_v)

    def chunk(j, carry):
        pltpu.async_copy(tab_hbm.at[idx_v.at[j]], rows_v, gsem).wait()
        pltpu.sync_copy(rows_v, out_hbm.at[blk0 + j])
        return carry

    lax.fori_loop(0, cpw, chunk, 0)


def kernel(input_, weight):
    bsz, hist = input_.shape
    B = bsz * hist
    nblk = B // C
    idx = input_.reshape(nblk, C).astype(jnp.int32)
    cpw = nblk // NW
    mesh = plsc.VectorSubcoreMesh(core_axis_name="c", subcore_axis_name="s")
    k = pl.kernel(
        _emb_body,
        mesh=mesh,
        out_type=jax.ShapeDtypeStruct((nblk, C, D), jnp.float32),
        scratch_types=[
            pltpu.VMEM((cpw, C), jnp.int32),
            pltpu.VMEM((C, D), jnp.float32),
            pltpu.SemaphoreType.DMA,
        ],
    )
    out = k(idx, weight)
    return out.reshape(bsz, hist, D)


# speedup vs baseline: 1.6837x; 1.6837x over previous
import functools

import jax
import jax.numpy as jnp
from jax import lax
from jax.experimental import pallas as pl
from jax.experimental.pallas import tpu as pltpu
from jax.experimental.pallas import tpu_sc as plsc

D = 64
C = 128
NC = 2
NS = 16
NW = NC * NS


def _emb_body(idx_hbm, tab_hbm, out_hbm, idx_v, rows_v, gsem):
    wid = lax.axis_index("s") * NC + lax.axis_index("c")
    cpw = idx_hbm.shape[0] // NW
    blk0 = wid * cpw
    pltpu.sync_copy(idx_hbm.at[pl.ds(blk0, cpw)], idx_v)

    def chunk(j, carry):
        pltpu.async_copy(tab_hbm.at[idx_v.at[j]], rows_v, gsem).wait()
        pltpu.sync_copy(rows_v, out_hbm.at[blk0 + j])
        return carry

    lax.fori_loop(0, cpw, chunk, 0)


def kernel(input_, weight):
    bsz, hist = input_.shape
    B = bsz * hist
    nblk = B // C
    idx = input_.reshape(nblk, C).astype(jnp.int32)
    cpw = nblk // NW
    mesh = plsc.VectorSubcoreMesh(core_axis_name="c", subcore_axis_name="s")
    k = pl.kernel(
        _emb_body,
        mesh=mesh,
        out_type=jax.ShapeDtypeStruct((nblk, C, D), jnp.float32),
        scratch_types=[
            pltpu.VMEM((cpw, C), jnp.int32),
            pltpu.VMEM((C, D), jnp.float32),
            pltpu.SemaphoreType.DMA,
        ],
        compiler_params=pltpu.CompilerParams(use_tc_tiling_on_sc=False),
    )
    out = k(idx, weight)
    return out.reshape(bsz, hist, D)


# speedup vs baseline: 1.8842x; 1.1190x over previous
import jax
import jax.numpy as jnp
from jax import lax
from jax.experimental import pallas as pl
from jax.experimental.pallas import tpu as pltpu
from jax.experimental.pallas import tpu_sc as plsc

D = 64
C = 128
NC = 2
NS = 16
NW = NC * NS
NBUF = 8


def _emb_body(idx_hbm, tab_hbm, out_hbm, idx_v, rows_v, gsems, osems):
    wid = lax.axis_index("s") * NC + lax.axis_index("c")
    cpw = idx_hbm.shape[0] // NW
    ng = cpw // NBUF
    blk0 = wid * cpw
    pltpu.sync_copy(idx_hbm.at[pl.ds(blk0, cpw)], idx_v)

    def gather(j, b):
        return pltpu.make_async_copy(
            tab_hbm.at[idx_v.at[j]], rows_v.at[b], gsems.at[b])

    def outcopy(j, b):
        return pltpu.make_async_copy(
            rows_v.at[b], out_hbm.at[blk0 + j], osems.at[b])

    for b in range(NBUF):
        gather(b, b).start()

    def group(g, carry):
        for b in range(NBUF):
            j = g * NBUF + b
            gather(j, b).wait()
            outcopy(j, b).start()
        for b in range(NBUF):
            j = g * NBUF + b
            outcopy(j, b).wait()
            gather(j + NBUF, b).start()
        return carry

    lax.fori_loop(0, ng - 1, group, 0)

    last = (ng - 1) * NBUF
    for b in range(NBUF):
        gather(last + b, b).wait()
        outcopy(last + b, b).start()
    for b in range(NBUF):
        outcopy(last + b, b).wait()


def kernel(input_, weight):
    bsz, hist = input_.shape
    B = bsz * hist
    nblk = B // C
    idx = input_.reshape(nblk, C).astype(jnp.int32)
    cpw = nblk // NW
    mesh = plsc.VectorSubcoreMesh(core_axis_name="c", subcore_axis_name="s")
    k = pl.kernel(
        _emb_body,
        mesh=mesh,
        out_type=jax.ShapeDtypeStruct((nblk, C, D), jnp.float32),
        scratch_types=[
            pltpu.VMEM((cpw, C), jnp.int32),
            pltpu.VMEM((NBUF, C, D), jnp.float32),
            pltpu.SemaphoreType.DMA((NBUF,)),
            pltpu.SemaphoreType.DMA((NBUF,)),
        ],
        compiler_params=pltpu.CompilerParams(use_tc_tiling_on_sc=False),
    )
    out = k(idx, weight)
    return out.reshape(bsz, hist, D)
